# baseline (speedup 1.0000x reference)
import functools

import jax
import jax.numpy as jnp
from jax import lax
from jax.experimental import pallas as pl
from jax.experimental.pallas import tpu as pltpu
from jax.experimental.pallas import tpu_sc as plsc

_D = 64
_B = 16384


@functools.cache
def _build(nc, ns):
    nw = nc * ns
    b_per_w = _B // nw
    mesh = plsc.VectorSubcoreMesh(core_axis_name="c", subcore_axis_name="s")

    @functools.partial(
        pl.kernel,
        mesh=mesh,
        compiler_params=pltpu.CompilerParams(use_tc_tiling_on_sc=False),
        out_type=[
            jax.ShapeDtypeStruct((_B, _D), jnp.float32),
            jax.ShapeDtypeStruct((_B, _D), jnp.float32),
            jax.ShapeDtypeStruct((_B, _D), jnp.float32),
        ],
        scratch_types=[
            pltpu.VMEM((b_per_w,), jnp.int32),
            pltpu.VMEM((b_per_w,), jnp.int32),
            pltpu.VMEM((b_per_w,), jnp.int32),
            pltpu.VMEM((b_per_w, _D), jnp.float32),
            pltpu.VMEM((b_per_w, _D), jnp.float32),
            pltpu.VMEM((b_per_w, _D), jnp.float32),
            pltpu.SemaphoreType.DMA,
            pltpu.SemaphoreType.DMA,
            pltpu.SemaphoreType.DMA,
        ],
    )
    def k(users_hbm, pos_hbm, neg_hbm, ut_hbm, it_hbm,
          out_u, out_p, out_n,
          idx_u, idx_p, idx_n, rows_u, rows_p, rows_n,
          sem_u, sem_p, sem_n):
        wid = lax.axis_index("s") * nc + lax.axis_index("c")
        base = wid * b_per_w
        pltpu.sync_copy(users_hbm.at[pl.ds(base, b_per_w)], idx_u)
        pltpu.sync_copy(pos_hbm.at[pl.ds(base, b_per_w)], idx_p)
        pltpu.sync_copy(neg_hbm.at[pl.ds(base, b_per_w)], idx_n)
        cu = pltpu.async_copy(ut_hbm.at[idx_u], rows_u, sem_u)
        cp = pltpu.async_copy(it_hbm.at[idx_p], rows_p, sem_p)
        cn = pltpu.async_copy(it_hbm.at[idx_n], rows_n, sem_n)
        cu.wait()
        pltpu.sync_copy(rows_u, out_u.at[pl.ds(base, b_per_w)])
        cp.wait()
        pltpu.sync_copy(rows_p, out_p.at[pl.ds(base, b_per_w)])
        cn.wait()
        pltpu.sync_copy(rows_n, out_n.at[pl.ds(base, b_per_w)])

    return k


def kernel(users, pos_items, neg_items, user_table, item_table):
    info = plsc.get_sparse_core_info()
    k = _build(info.num_cores, info.num_subcores)
    out = k(users, pos_items, neg_items, user_table, item_table)
    return tuple(out)


# speedup vs baseline: 1.5642x; 1.5642x over previous
import functools

import jax
import jax.numpy as jnp
from jax import lax
from jax.experimental import pallas as pl
from jax.experimental.pallas import tpu as pltpu
from jax.experimental.pallas import tpu_sc as plsc

_D = 64
_B = 16384


@functools.cache
def _build(nc, ns):
    nw = nc * ns
    b_per_w = _B // nw
    nblk = b_per_w // 8
    half = b_per_w // 2
    hblk = half // 8
    mesh = plsc.VectorSubcoreMesh(core_axis_name="c", subcore_axis_name="s")

    out_sds = jax.ShapeDtypeStruct((_B // 8, 8, _D), jnp.float32)

    @functools.partial(
        pl.kernel,
        mesh=mesh,
        out_type=[out_sds, out_sds, out_sds],
        scratch_types=[
            pltpu.VMEM((b_per_w,), jnp.int32),
            pltpu.VMEM((hblk, 8, _D), jnp.float32),
            pltpu.VMEM((hblk, 8, _D), jnp.float32),
            pltpu.SemaphoreType.DMA,
            pltpu.SemaphoreType.DMA,
            pltpu.SemaphoreType.DMA,
            pltpu.SemaphoreType.DMA,
        ],
    )
    def k(users_hbm, pos_hbm, neg_hbm, ut_hbm, it_hbm,
          out_u, out_p, out_n,
          idx_v, rows_a, rows_b, sem_a, sem_b, wsem_a, wsem_b):
        wid = lax.axis_index("s") * nc + lax.axis_index("c")
        base = wid * b_per_w

        rows = (rows_a, rows_b)
        gsem = (sem_a, sem_b)
        wsem = (wsem_a, wsem_b)
        writes = [None, None]

        step = 0
        for idx_hbm, tbl_hbm, out_hbm in (
            (users_hbm, ut_hbm, out_u),
            (pos_hbm, it_hbm, out_p),
            (neg_hbm, it_hbm, out_n),
        ):
            pltpu.sync_copy(idx_hbm.at[pl.ds(base, b_per_w)], idx_v)

            for h in range(2):
                p = step % 2
                step += 1
                if writes[p] is not None:
                    writes[p].wait()

                def fire(g, _, off=h * half, tbl_hbm=tbl_hbm,
                         rbuf=rows[p], sem=gsem[p]):
                    v = idx_v[pl.ds(off + g * 16, 16)]
                    for l in range(16):
                        pltpu.async_copy(
                            tbl_hbm.at[v[l]],
                            rbuf.at[g * 2 + l // 8, l % 8],
                            sem,
                        )
                    return _

                lax.fori_loop(0, half // 16, fire, 0)

                dst = out_hbm.at[pl.ds(wid * nblk + h * hblk, hblk)]
                pltpu.make_async_copy(dst, rows[p], gsem[p]).wait()
                writes[p] = pltpu.async_copy(rows[p], dst, wsem[p])

        writes[0].wait()
        writes[1].wait()

    return k


def kernel(users, pos_items, neg_items, user_table, item_table):
    info = plsc.get_sparse_core_info()
    k = _build(info.num_cores, info.num_subcores)
    ou, op, on = k(users, pos_items, neg_items, user_table, item_table)
    return (ou.reshape(_B, _D), op.reshape(_B, _D), on.reshape(_B, _D))
